# Initial kernel scaffold; baseline (speedup 1.0000x reference)
import jax
import jax.numpy as jnp
from jax.experimental import pallas as pl


def kernel(x, neutral_idx, W):
    raise NotImplementedError("write your pallas kernel here")


# speedup vs baseline: 1.0190x; 1.0190x over previous
import functools

import jax
import jax.numpy as jnp
from jax import lax
from jax.experimental import pallas as pl
from jax.experimental.pallas import tpu as pltpu
from jax.experimental.pallas import tpu_sc as plsc

NC = 2
NS = 16
L = 16
NW = NC * NS

LN2 = 0.6931471805599453


@functools.partial(jax.jit, static_argnames=("b_per_w",))
def _gain(x, nidx, table, b_per_w):
    B = x.shape[0]

    @functools.partial(
        pl.kernel,
        out_type=jax.ShapeDtypeStruct((B,), jnp.float32),
        mesh=plsc.VectorSubcoreMesh(core_axis_name="c", subcore_axis_name="s"),
        scratch_types=[
            pltpu.VMEM((b_per_w,), jnp.int32),
            pltpu.VMEM((b_per_w,), jnp.float32),
            pltpu.VMEM((L,), jnp.int32),
            pltpu.VMEM((L,), jnp.float32),
            pltpu.SemaphoreType.DMA,
            pltpu.SemaphoreType.DMA,
        ],
    )
    def body(x_hbm, nidx_hbm, table_hbm, out_hbm, idx_v, rows_v, nidx_v, nval_v, sem_r, sem_n):
        wid = lax.axis_index("s") * NC + lax.axis_index("c")
        base = wid * b_per_w
        pltpu.sync_copy(x_hbm.at[pl.ds(base, b_per_w)], idx_v)
        pltpu.sync_copy(nidx_hbm, nidx_v)
        cp_rows = pltpu.async_copy(table_hbm.at[idx_v], rows_v, sem_r)
        cp_n = pltpu.async_copy(table_hbm.at[nidx_v], nval_v, sem_n)
        cp_n.wait()
        cp_rows.wait()
        neutral = nval_v[...]
        for i in range(b_per_w // L):
            sl = pl.ds(i * L, L)
            rows_v[sl] = jnp.exp((rows_v[sl] - neutral) * LN2)
        pltpu.sync_copy(rows_v, out_hbm.at[pl.ds(base, b_per_w)])

    return body(x, nidx, table)


def kernel(x, neutral_idx, W):
    B = x.shape[0]
    table = W.reshape(W.shape[0])
    nidx = jnp.full((L,), neutral_idx, dtype=jnp.int32)
    out = _gain(x, nidx, table, B // NW)
    return out.reshape(B, 1)


# speedup vs baseline: 1.0229x; 1.0039x over previous
import functools

import jax
import jax.numpy as jnp
from jax import lax
from jax.experimental import pallas as pl
from jax.experimental.pallas import tpu as pltpu
from jax.experimental.pallas import tpu_sc as plsc

NC = 2
NS = 16
L = 16
NW = NC * NS

LN2 = 0.6931471805599453


@functools.partial(jax.jit, static_argnames=("b_per_w",))
def _gain(x, nidx, table, b_per_w):
    B = x.shape[0]

    @functools.partial(
        pl.kernel,
        out_type=jax.ShapeDtypeStruct((B,), jnp.float32),
        mesh=plsc.VectorSubcoreMesh(core_axis_name="c", subcore_axis_name="s"),
        scratch_types=[
            pltpu.VMEM((b_per_w + L,), jnp.int32),
            pltpu.VMEM((b_per_w + L,), jnp.float32),
            pltpu.SemaphoreType.DMA,
            pltpu.SemaphoreType.DMA,
        ],
    )
    def body(x_hbm, nidx_hbm, table_hbm, out_hbm, idx_v, rows_v, sem_l, sem_g):
        wid = lax.axis_index("s") * NC + lax.axis_index("c")
        base = wid * b_per_w
        cp_x = pltpu.async_copy(
            x_hbm.at[pl.ds(base, b_per_w)], idx_v.at[pl.ds(0, b_per_w)], sem_l)
        cp_n = pltpu.async_copy(nidx_hbm, idx_v.at[pl.ds(b_per_w, L)], sem_l)
        cp_x.wait()
        cp_n.wait()
        cp_g = pltpu.async_copy(table_hbm.at[idx_v], rows_v, sem_g)
        cp_g.wait()
        neutral = rows_v[pl.ds(b_per_w, L)]
        for i in range(b_per_w // L):
            sl = pl.ds(i * L, L)
            rows_v[sl] = jnp.exp((rows_v[sl] - neutral) * LN2)
        pltpu.sync_copy(rows_v.at[pl.ds(0, b_per_w)], out_hbm.at[pl.ds(base, b_per_w)])

    return body(x, nidx, table)


def kernel(x, neutral_idx, W):
    B = x.shape[0]
    table = W.reshape(W.shape[0])
    nidx = jnp.full((L,), neutral_idx, dtype=jnp.int32)
    out = _gain(x, nidx, table, B // NW)
    return out.reshape(B, 1)


# speedup vs baseline: 1.3377x; 1.3078x over previous
import functools

import jax
import jax.numpy as jnp
from jax import lax
from jax.experimental import pallas as pl
from jax.experimental.pallas import tpu as pltpu
from jax.experimental.pallas import tpu_sc as plsc

NC = 2
NS = 16
L = 16
NW = NC * NS

LN2 = 0.6931471805599453
RW = 128
TAIL = 1024


@functools.partial(jax.jit, static_argnames=("b_per_w", "v"))
def _gain(x, nidx, main, tail, b_per_w, v):
    B = x.shape[0]
    n_rows = main.shape[0]
    main_len = n_rows * RW
    tail_start = v - TAIL
    nb = b_per_w + L

    @functools.partial(
        pl.kernel,
        out_type=jax.ShapeDtypeStruct((B,), jnp.float32),
        mesh=plsc.VectorSubcoreMesh(core_axis_name="c", subcore_axis_name="s"),
        compiler_params=pltpu.CompilerParams(needs_layout_passes=False),
        scratch_types=[
            pltpu.VMEM((nb,), jnp.int32),
            pltpu.VMEM((nb,), jnp.int32),
            pltpu.VMEM((nb, RW), jnp.float32),
            pltpu.VMEM((TAIL,), jnp.float32),
            pltpu.VMEM((b_per_w,), jnp.float32),
            pltpu.SemaphoreType.DMA,
            pltpu.SemaphoreType.DMA,
        ],
    )
    def body(x_hbm, nidx_hbm, main_hbm, tail_hbm, out_hbm,
             idx_v, row_v, rows_v, tail_v, out_v, sem_l, sem_g):
        wid = lax.axis_index("s") * NC + lax.axis_index("c")
        base = wid * b_per_w
        cp_x = pltpu.async_copy(
            x_hbm.at[pl.ds(base, b_per_w)], idx_v.at[pl.ds(0, b_per_w)], sem_l)
        cp_n = pltpu.async_copy(nidx_hbm, idx_v.at[pl.ds(b_per_w, L)], sem_l)
        cp_t = pltpu.async_copy(tail_hbm, tail_v, sem_l)
        cp_x.wait()
        cp_n.wait()
        n_row_max = jnp.full((L,), n_rows - 1, jnp.int32)
        for i in range(nb // L):
            sl = pl.ds(i * L, L)
            c = idx_v[sl]
            row_v[sl] = jnp.minimum(lax.shift_right_logical(c, 7), n_row_max)
        cp_g = pltpu.async_copy(main_hbm.at[row_v], rows_v, sem_g)
        cp_t.wait()
        cp_g.wait()

        lanes = lax.iota(jnp.int32, L)
        mask7 = jnp.full((L,), RW - 1, jnp.int32)
        zero = jnp.zeros((L,), jnp.int32)

        def pick(i):
            c = idx_v[pl.ds(i * L, L)]
            v_main = plsc.load_gather(rows_v, [lanes + i * L, c & mask7])
            t_idx = jnp.maximum(c - tail_start, zero)
            v_tail = plsc.load_gather(tail_v, [t_idx])
            return jnp.where(c >= main_len, v_tail, v_main)

        neutral = pick(b_per_w // L)
        for i in range(b_per_w // L):
            out_v[pl.ds(i * L, L)] = jnp.exp((pick(i) - neutral) * LN2)
        pltpu.sync_copy(out_v, out_hbm.at[pl.ds(base, b_per_w)])

    return body(x, nidx, main, tail)


def kernel(x, neutral_idx, W):
    B = x.shape[0]
    V = W.shape[0]
    n_rows = V // RW // 8 * 8
    main = lax.slice(W, (0, 0), (n_rows * RW, 1)).reshape(n_rows, RW)
    tail = lax.slice(W, (V - TAIL, 0), (V, 1)).reshape(TAIL)
    nidx = jnp.full((L,), neutral_idx, dtype=jnp.int32)
    out = _gain(x, nidx, main, tail, B // NW, V)
    return out.reshape(B, 1)


# speedup vs baseline: 1.6067x; 1.2011x over previous
import functools

import jax
import jax.numpy as jnp
from jax import lax
from jax.experimental import pallas as pl
from jax.experimental.pallas import tpu as pltpu
from jax.experimental.pallas import tpu_sc as plsc

NC = 2
NS = 16
L = 16
NW = NC * NS

LN2 = 0.6931471805599453
RW = 128
TAIL = 1024
FLAT_GRID = 16


def _flatten(main):
    n_rows = main.shape[0]
    flat = n_rows * RW
    rows_blk = n_rows // FLAT_GRID

    def body(m_ref, o_ref):
        o_ref[...] = m_ref[...].reshape(rows_blk * RW)

    return pl.pallas_call(
        body,
        grid=(FLAT_GRID,),
        in_specs=[pl.BlockSpec((rows_blk, RW), lambda i: (i, 0))],
        out_specs=pl.BlockSpec((rows_blk * RW,), lambda i: (i,)),
        out_shape=jax.ShapeDtypeStruct((flat,), jnp.float32),
    )(main)


@functools.partial(jax.jit, static_argnames=("b_per_w", "v"))
def _gain(x, nidx, main, tail, b_per_w, v):
    B = x.shape[0]
    main_len = main.shape[0]
    tail_start = v - TAIL
    nb = b_per_w + L

    @functools.partial(
        pl.kernel,
        out_type=jax.ShapeDtypeStruct((B,), jnp.float32),
        mesh=plsc.VectorSubcoreMesh(core_axis_name="c", subcore_axis_name="s"),
        compiler_params=pltpu.CompilerParams(needs_layout_passes=False),
        scratch_types=[
            pltpu.VMEM((nb,), jnp.int32),
            pltpu.VMEM((nb,), jnp.int32),
            pltpu.VMEM((nb,), jnp.float32),
            pltpu.VMEM((TAIL,), jnp.float32),
            pltpu.VMEM((b_per_w,), jnp.float32),
            pltpu.SemaphoreType.DMA,
            pltpu.SemaphoreType.DMA,
        ],
    )
    def body(x_hbm, nidx_hbm, main_hbm, tail_hbm, out_hbm,
             idx_v, cl_v, rows_v, tail_v, out_v, sem_l, sem_g):
        wid = lax.axis_index("s") * NC + lax.axis_index("c")
        base = wid * b_per_w
        cp_x = pltpu.async_copy(
            x_hbm.at[pl.ds(base, b_per_w)], idx_v.at[pl.ds(0, b_per_w)], sem_l)
        cp_n = pltpu.async_copy(nidx_hbm, idx_v.at[pl.ds(b_per_w, L)], sem_l)
        cp_t = pltpu.async_copy(tail_hbm, tail_v, sem_l)
        cp_x.wait()
        cp_n.wait()
        max_i = jnp.full((L,), main_len - 1, jnp.int32)
        for i in range(nb // L):
            sl = pl.ds(i * L, L)
            cl_v[sl] = jnp.minimum(idx_v[sl], max_i)
        cp_g = pltpu.async_copy(main_hbm.at[cl_v], rows_v, sem_g)
        cp_t.wait()
        cp_g.wait()

        zero = jnp.zeros((L,), jnp.int32)

        def pick(i):
            sl = pl.ds(i * L, L)
            c = idx_v[sl]
            t_idx = jnp.maximum(c - tail_start, zero)
            v_tail = plsc.load_gather(tail_v, [t_idx])
            return jnp.where(c >= main_len, v_tail, rows_v[sl])

        neutral = pick(b_per_w // L)
        for i in range(b_per_w // L):
            out_v[pl.ds(i * L, L)] = jnp.exp((pick(i) - neutral) * LN2)
        pltpu.sync_copy(out_v, out_hbm.at[pl.ds(base, b_per_w)])

    return body(x, nidx, main, tail)


def kernel(x, neutral_idx, W):
    B = x.shape[0]
    V = W.shape[0]
    n_rows = V // RW // 8 * 8
    main = _flatten(lax.slice(W, (0, 0), (n_rows * RW, 1)).reshape(n_rows, RW))
    tail = lax.slice(W, (V - TAIL, 0), (V, 1)).reshape(TAIL)
    nidx = jnp.full((L,), neutral_idx, dtype=jnp.int32)
    out = _gain(x, nidx, main, tail, B // NW, V)
    return out.reshape(B, 1)
